# baseline (speedup 1.0000x reference)
import functools

import jax
import jax.numpy as jnp
from jax import lax
from jax.experimental import pallas as pl
from jax.experimental.pallas import tpu as pltpu
from jax.experimental.pallas import tpu_sc as plsc

_NBUF = 8


@functools.cache
def _make_gather(batch, seq, V, D):
    info = plsc.get_sparse_core_info()
    nw = info.num_cores * info.num_subcores
    rows_per_w = batch // nw
    n_rings = rows_per_w // _NBUF
    mesh = plsc.VectorSubcoreMesh(core_axis_name="c", subcore_axis_name="s")

    @functools.partial(
        pl.kernel,
        out_type=jax.ShapeDtypeStruct((batch, seq, D), jnp.float32),
        mesh=mesh,
        compiler_params=pltpu.CompilerParams(use_tc_tiling_on_sc=False),
        scratch_types=[
            pltpu.VMEM((rows_per_w, seq), jnp.int32),
            pltpu.VMEM((_NBUF, seq, D), jnp.float32),
            pltpu.SemaphoreType.DMA((_NBUF,)),
            pltpu.SemaphoreType.DMA((_NBUF,)),
        ],
    )
    def gather_kernel(idx_hbm, table_hbm, out_hbm, idx_v, rows_v, gsem, wsem):
        wid = lax.axis_index("s") * info.num_cores + lax.axis_index("c")
        rbase = wid * rows_per_w
        pltpu.sync_copy(idx_hbm.at[pl.ds(rbase, rows_per_w)], idx_v)

        def fire(j, b):
            pltpu.async_copy(
                table_hbm.at[idx_v.at[j]], rows_v.at[b], gsem.at[b]
            )

        def drain_fire_wb(j, b):
            pltpu.make_async_copy(
                table_hbm.at[idx_v.at[0]], rows_v.at[b], gsem.at[b]
            ).wait()
            pltpu.async_copy(rows_v.at[b], out_hbm.at[rbase + j], wsem.at[b])

        def wait_wb(j, b):
            pltpu.make_async_copy(
                rows_v.at[b], out_hbm.at[rbase + j], wsem.at[b]
            ).wait()

        for b in range(_NBUF):
            fire(b, b)

        def ring_body(g, carry):
            jbase = g * _NBUF
            for b in range(_NBUF):
                drain_fire_wb(jbase + b, b)
            for b in range(_NBUF):
                wait_wb(jbase + b, b)
                fire(jbase + _NBUF + b, b)
            return carry

        lax.fori_loop(0, n_rings - 1, ring_body, 0)

        jbase = (n_rings - 1) * _NBUF
        for b in range(_NBUF):
            drain_fire_wb(jbase + b, b)
        for b in range(_NBUF):
            wait_wb(jbase + b, b)

    return gather_kernel


@jax.jit
def kernel(inputs, table):
    batch, seq = inputs.shape
    vocab, embed = table.shape
    return _make_gather(batch, seq, vocab, embed)(inputs, table)


# speedup vs baseline: 3.4143x; 3.4143x over previous
import functools

import jax
import jax.numpy as jnp
from jax import lax
from jax.experimental import pallas as pl
from jax.experimental.pallas import tpu as pltpu
from jax.experimental.pallas import tpu_sc as plsc

_CHUNK = 128
_NBUF = 4


@functools.cache
def _make_gather(B, V, D):
    info = plsc.get_sparse_core_info()
    nw = info.num_cores * info.num_subcores
    b_per_w = B // nw
    n_chunks = b_per_w // _CHUNK
    n_rings = n_chunks // _NBUF
    mesh = plsc.VectorSubcoreMesh(core_axis_name="c", subcore_axis_name="s")

    @functools.partial(
        pl.kernel,
        out_type=jax.ShapeDtypeStruct((B, D), jnp.float32),
        mesh=mesh,
        compiler_params=pltpu.CompilerParams(use_tc_tiling_on_sc=False),
        scratch_types=[
            pltpu.VMEM((b_per_w,), jnp.int32),
            pltpu.VMEM((_NBUF, _CHUNK, D), jnp.float32),
            pltpu.SemaphoreType.DMA((_NBUF,)),
            pltpu.SemaphoreType.DMA((_NBUF,)),
        ],
    )
    def gather_kernel(idx_hbm, table_hbm, out_hbm, idx_v, rows_v, gsem, wsem):
        wid = lax.axis_index("s") * info.num_cores + lax.axis_index("c")
        base = wid * b_per_w
        pltpu.sync_copy(idx_hbm.at[pl.ds(base, b_per_w)], idx_v)

        def fire(j, b):
            pltpu.async_copy(
                table_hbm.at[idx_v.at[pl.ds(j * _CHUNK, _CHUNK)]],
                rows_v.at[b],
                gsem.at[b],
            )

        def drain_fire_wb(j, b):
            pltpu.make_async_copy(
                table_hbm.at[idx_v.at[pl.ds(0, _CHUNK)]], rows_v.at[b], gsem.at[b]
            ).wait()
            pltpu.async_copy(
                rows_v.at[b], out_hbm.at[pl.ds(base + j * _CHUNK, _CHUNK)], wsem.at[b]
            )

        def wait_wb(j, b):
            pltpu.make_async_copy(
                rows_v.at[b], out_hbm.at[pl.ds(base + j * _CHUNK, _CHUNK)], wsem.at[b]
            ).wait()

        for b in range(_NBUF):
            fire(b, b)

        def ring_body(g, carry):
            jbase = g * _NBUF
            for b in range(_NBUF):
                drain_fire_wb(jbase + b, b)
            for b in range(_NBUF):
                wait_wb(jbase + b, b)
                fire(jbase + _NBUF + b, b)
            return carry

        lax.fori_loop(0, n_rings - 1, ring_body, 0)

        jbase = (n_rings - 1) * _NBUF
        for b in range(_NBUF):
            drain_fire_wb(jbase + b, b)
        for b in range(_NBUF):
            wait_wb(jbase + b, b)

    return gather_kernel


@jax.jit
def kernel(inputs, table):
    batch, seq = inputs.shape
    vocab, embed = table.shape
    idx = inputs.T.reshape(-1)
    out = _make_gather(idx.shape[0], vocab, embed)(idx, table)
    return out.reshape(seq, batch, embed).transpose(1, 0, 2)


# speedup vs baseline: 3.4269x; 1.0037x over previous
import functools

import jax
import jax.numpy as jnp
from jax import lax
from jax.experimental import pallas as pl
from jax.experimental.pallas import tpu as pltpu
from jax.experimental.pallas import tpu_sc as plsc

_CHUNK = 200
_NBUF = 4


@functools.cache
def _make_gather(B, V, D):
    info = plsc.get_sparse_core_info()
    nw = info.num_cores * info.num_subcores
    b_per_w = B // nw
    n_chunks = b_per_w // _CHUNK
    n_rings = n_chunks // _NBUF
    mesh = plsc.VectorSubcoreMesh(core_axis_name="c", subcore_axis_name="s")

    @functools.partial(
        pl.kernel,
        out_type=jax.ShapeDtypeStruct((B, D), jnp.float32),
        mesh=mesh,
        compiler_params=pltpu.CompilerParams(use_tc_tiling_on_sc=False),
        scratch_types=[
            pltpu.VMEM((b_per_w,), jnp.int32),
            pltpu.VMEM((_NBUF, _CHUNK, D), jnp.float32),
            pltpu.SemaphoreType.DMA((_NBUF,)),
            pltpu.SemaphoreType.DMA((_NBUF,)),
        ],
    )
    def gather_kernel(idx_hbm, table_hbm, out_hbm, idx_v, rows_v, gsem, wsem):
        wid = lax.axis_index("s") * info.num_cores + lax.axis_index("c")
        base = wid * b_per_w
        pltpu.sync_copy(idx_hbm.at[pl.ds(base, b_per_w)], idx_v)

        def fire(j, b):
            pltpu.async_copy(
                table_hbm.at[idx_v.at[pl.ds(j * _CHUNK, _CHUNK)]],
                rows_v.at[b],
                gsem.at[b],
            )

        def drain_fire_wb(j, b):
            pltpu.make_async_copy(
                table_hbm.at[idx_v.at[pl.ds(0, _CHUNK)]], rows_v.at[b], gsem.at[b]
            ).wait()
            pltpu.async_copy(
                rows_v.at[b], out_hbm.at[pl.ds(base + j * _CHUNK, _CHUNK)], wsem.at[b]
            )

        def wait_wb(j, b):
            pltpu.make_async_copy(
                rows_v.at[b], out_hbm.at[pl.ds(base + j * _CHUNK, _CHUNK)], wsem.at[b]
            ).wait()

        for b in range(_NBUF):
            fire(b, b)

        def ring_body(g, carry):
            jbase = g * _NBUF
            for b in range(_NBUF):
                drain_fire_wb(jbase + b, b)
            for b in range(_NBUF):
                wait_wb(jbase + b, b)
                fire(jbase + _NBUF + b, b)
            return carry

        lax.fori_loop(0, n_rings - 1, ring_body, 0)

        jbase = (n_rings - 1) * _NBUF
        for b in range(_NBUF):
            drain_fire_wb(jbase + b, b)
        for b in range(_NBUF):
            wait_wb(jbase + b, b)

    return gather_kernel


@jax.jit
def kernel(inputs, table):
    batch, seq = inputs.shape
    vocab, embed = table.shape
    idx = inputs.T.reshape(-1)
    out = _make_gather(idx.shape[0], vocab, embed)(idx, table)
    return out.reshape(seq, batch, embed).transpose(1, 0, 2)
